# Initial kernel scaffold; baseline (speedup 1.0000x reference)
import jax
import jax.numpy as jnp
from jax.experimental import pallas as pl


def kernel(atom_fea, nbr_fea, nbr_fea_idx, bond_weights_ag, W_full, b_full, g1, b1, g2, b2, W_proj, b_proj):
    raise NotImplementedError("write your pallas kernel here")


# speedup vs baseline: 2.7526x; 2.7526x over previous
import functools

import jax
import jax.numpy as jnp
from jax import lax
from jax.experimental import pallas as pl
from jax.experimental.pallas import tpu as pltpu
from jax.experimental.pallas import tpu_sc as plsc

_EPS = 1e-5


def _softplus(x):
    return jnp.maximum(x, 0.0) + jnp.log1p(jnp.exp(-jnp.abs(x)))


def _sigmoid(x):
    return 1.0 / (1.0 + jnp.exp(-x))


def _sc_gather(table, idx_flat, chunk=200):
    n_rows, d = table.shape
    b = idx_flat.shape[0]
    info = plsc.get_sparse_core_info()
    nw = info.num_cores * info.num_subcores
    per_w = b // nw
    assert per_w * nw == b and per_w % (2 * chunk) == 0 and chunk % 8 == 0
    n_half = per_w // chunk // 2
    mesh = plsc.VectorSubcoreMesh(core_axis_name="c", subcore_axis_name="s")

    @functools.partial(
        pl.kernel,
        mesh=mesh,
        out_type=jax.ShapeDtypeStruct((b, d), table.dtype),
        scratch_types=[
            pltpu.VMEM((chunk,), jnp.int32),
            pltpu.VMEM((chunk,), jnp.int32),
            pltpu.VMEM((chunk, d), table.dtype),
            pltpu.VMEM((chunk, d), table.dtype),
            pltpu.SemaphoreType.DMA,
            pltpu.SemaphoreType.DMA,
            pltpu.SemaphoreType.DMA,
        ],
    )
    def k(table_hbm, idx_hbm, out_hbm, idx_v0, idx_v1, rows_v0, rows_v1,
          sem_g, sem_o0, sem_o1):
        wid = lax.axis_index("s") * info.num_cores + lax.axis_index("c")
        base = pl.multiple_of(wid * per_w, 8)
        idx_v = (idx_v0, idx_v1)
        rows_v = (rows_v0, rows_v1)
        sem_o = (sem_o0, sem_o1)

        def body(j, _):
            for sl in range(2):
                ci = 2 * j + sl
                off = pl.multiple_of(base + ci * chunk, 8)

                @pl.when(j > 0)
                def _drain():
                    pltpu.make_async_copy(
                        rows_v[sl], out_hbm.at[pl.ds(off, chunk)],
                        sem_o[sl]).wait()

                pltpu.sync_copy(idx_hbm.at[pl.ds(off, chunk)], idx_v[sl])
                pltpu.async_copy(table_hbm.at[idx_v[sl]], rows_v[sl],
                                 sem_g).wait()
                pltpu.async_copy(rows_v[sl],
                                 out_hbm.at[pl.ds(off, chunk)], sem_o[sl])
            return ()

        lax.fori_loop(0, n_half, body, (), unroll=False)
        for sl in range(2):
            pltpu.make_async_copy(rows_v[sl],
                                  out_hbm.at[pl.ds(base, chunk)],
                                  sem_o[sl]).wait()

    return k(table, idx_flat)


def _center_body(atom_ref, wc_ref, bf_ref, out_ref):
    out_ref[...] = (
        jnp.dot(atom_ref[...], wc_ref[...], preferred_element_type=jnp.float32)
        + bf_ref[...]
    )


def _stats_body(g_ref, q_ref, a_ref, wg_ref, wn_ref, stats_ref):
    step = pl.program_id(0)

    @pl.when(step == 0)
    def _init():
        stats_ref[...] = jnp.zeros_like(stats_ref)

    a = a_ref[...]
    x2 = (
        jnp.dot(g_ref[...], wg_ref[...], preferred_element_type=jnp.float32)
        + jnp.dot(q_ref[...], wn_ref[...], preferred_element_type=jnp.float32)
    )
    ba, hf = a.shape
    m = x2.shape[0] // ba
    t = jnp.sum(x2.reshape(ba, m, hf), axis=1)
    s = jnp.sum(x2, axis=0) + m * jnp.sum(a, axis=0)
    sq = (
        jnp.sum(x2 * x2, axis=0)
        + 2.0 * jnp.sum(a * t, axis=0)
        + m * jnp.sum(a * a, axis=0)
    )
    stats_ref[0:1, :] += s.reshape(1, hf)
    stats_ref[1:2, :] += sq.reshape(1, hf)


def _apply_body(count, g_ref, q_ref, a_ref, w_ref, wg_ref, wn_ref,
                stats_ref, g1_ref, b1_ref, s_out_ref, st2_ref):
    step = pl.program_id(0)

    @pl.when(step == 0)
    def _init():
        st2_ref[...] = jnp.zeros_like(st2_ref)

    mu = stats_ref[0:1, :] / count
    ex2 = stats_ref[1:2, :] / count
    var = ex2 - mu * mu
    inv = lax.rsqrt(var + _EPS)
    scale = g1_ref[...] * inv
    shift = b1_ref[...] - mu * scale

    x2 = (
        jnp.dot(g_ref[...], wg_ref[...], preferred_element_type=jnp.float32)
        + jnp.dot(q_ref[...], wn_ref[...], preferred_element_type=jnp.float32)
    )
    a = a_ref[...]
    ba, hf = a.shape
    m = x2.shape[0] // ba
    af = hf // 2
    a2 = a * scale + shift
    y = x2 * scale
    y3 = y.reshape(ba, m, hf) + a2[:, None, :]
    f = _sigmoid(y3[:, :, :af])
    c = _softplus(y3[:, :, af:])
    w = w_ref[...]
    prod = f * c * (w * w)[:, :, None]
    s_blk = jnp.sum(prod, axis=1)
    s_out_ref[...] = s_blk
    st2_ref[0:1, :] += jnp.sum(s_blk, axis=0).reshape(1, af)
    st2_ref[1:2, :] += jnp.sum(s_blk * s_blk, axis=0).reshape(1, af)


def _final_body(count, s_ref, atom_ref, st2_ref, g2_ref, b2_ref,
                wp_ref, bp_ref, out_ref):
    mu = st2_ref[0:1, :] / count
    ex2 = st2_ref[1:2, :] / count
    var = ex2 - mu * mu
    inv = lax.rsqrt(var + _EPS)
    scale = g2_ref[...] * inv
    shift = b2_ref[...] - mu * scale
    h = _softplus(atom_ref[...] + s_ref[...] * scale + shift)
    out_ref[...] = (
        jnp.dot(h, wp_ref[...], preferred_element_type=jnp.float32)
        + bp_ref[...]
    )


def _tc_pipeline(atom_fea, g0, nbr_flat, bond_w, W_full, b_full,
                 g1, b1, g2, b2, W_proj, b_proj):
    n, af = atom_fea.shape
    nm = g0.shape[0]
    m = nm // n
    nf = nbr_flat.shape[1]
    hf = 2 * af

    wc = W_full[:af]
    wg = W_full[af:2 * af]
    wn = W_full[2 * af:]

    a_center = pl.pallas_call(
        _center_body,
        out_shape=jax.ShapeDtypeStruct((n, hf), jnp.float32),
    )(atom_fea, wc, b_full.reshape(1, hf))

    ba = 200
    be = ba * m
    nsteps = n // ba
    full = lambda shp: pl.BlockSpec(shp, lambda i: (0,) * len(shp))
    stats = pl.pallas_call(
        _stats_body,
        grid=(nsteps,),
        in_specs=[
            pl.BlockSpec((be, af), lambda i: (i, 0)),
            pl.BlockSpec((be, nf), lambda i: (i, 0)),
            pl.BlockSpec((ba, hf), lambda i: (i, 0)),
            full((af, hf)),
            full((nf, hf)),
        ],
        out_specs=pl.BlockSpec((8, hf), lambda i: (0, 0)),
        out_shape=jax.ShapeDtypeStruct((8, hf), jnp.float32),
    )(g0, nbr_flat, a_center, wg, wn)

    s_sum, st2 = pl.pallas_call(
        functools.partial(_apply_body, float(nm)),
        grid=(nsteps,),
        in_specs=[
            pl.BlockSpec((be, af), lambda i: (i, 0)),
            pl.BlockSpec((be, nf), lambda i: (i, 0)),
            pl.BlockSpec((ba, hf), lambda i: (i, 0)),
            pl.BlockSpec((ba, m), lambda i: (i, 0)),
            full((af, hf)),
            full((nf, hf)),
            full((8, hf)),
            full((1, hf)),
            full((1, hf)),
        ],
        out_specs=[
            pl.BlockSpec((ba, af), lambda i: (i, 0)),
            pl.BlockSpec((8, af), lambda i: (0, 0)),
        ],
        out_shape=[
            jax.ShapeDtypeStruct((n, af), jnp.float32),
            jax.ShapeDtypeStruct((8, af), jnp.float32),
        ],
    )(g0, nbr_flat, a_center, bond_w, wg, wn, stats,
      g1.reshape(1, hf), b1.reshape(1, hf))

    ba2 = 2000
    atom_out = pl.pallas_call(
        functools.partial(_final_body, float(n)),
        grid=(n // ba2,),
        in_specs=[
            pl.BlockSpec((ba2, af), lambda i: (i, 0)),
            pl.BlockSpec((ba2, af), lambda i: (i, 0)),
            full((8, af)),
            full((1, af)),
            full((1, af)),
            full((af, af)),
            full((1, af)),
        ],
        out_specs=pl.BlockSpec((ba2, af), lambda i: (i, 0)),
        out_shape=jax.ShapeDtypeStruct((n, af), jnp.float32),
    )(s_sum, atom_fea, st2, g2.reshape(1, af), b2.reshape(1, af),
      W_proj, b_proj.reshape(1, af))

    return atom_out


def kernel(atom_fea, nbr_fea, nbr_fea_idx, bond_weights_ag,
           W_full, b_full, g1, b1, g2, b2, W_proj, b_proj):
    n, m = nbr_fea_idx.shape
    nf = nbr_fea.shape[2]
    idx_flat = nbr_fea_idx.reshape(n * m).astype(jnp.int32)
    g0 = _sc_gather(atom_fea, idx_flat)
    nbr_flat = nbr_fea.reshape(n * m, nf)
    atom_out = _tc_pipeline(atom_fea, g0, nbr_flat, bond_weights_ag,
                            W_full, b_full, g1, b1, g2, b2, W_proj, b_proj)
    return atom_out, nbr_fea
